# baseline (speedup 1.0000x reference)
import functools

import jax
import jax.numpy as jnp
from jax import lax
from jax.experimental import pallas as pl
from jax.experimental.pallas import tpu as pltpu
from jax.experimental.pallas import tpu_sc as plsc

B, F, N = 16, 256, 2048
R, PNT = 16, 512
NF_HALF = F // 2
NCOPY = R + 3


def _sorter_body(w_ref, b_ref, x_ref, val_ref, id_ref):
    w = w_ref[...]
    xb = x_ref[...]
    val = jnp.dot(w, xb, preferred_element_type=jnp.float32)
    val = val + b_ref[...]
    val_ref[...] = val
    mx = jnp.max(val, axis=0, keepdims=True)
    iota = lax.broadcasted_iota(jnp.int32, (R, N), 0)
    ids = jnp.min(jnp.where(val == mx, iota, jnp.int32(2**30)), axis=0,
                  keepdims=True)
    id_ref[...] = ids


def _sorter(x, w2d, b2d):
    val, ids = pl.pallas_call(
        _sorter_body,
        grid=(B,),
        in_specs=[
            pl.BlockSpec((R, F), lambda b: (0, 0)),
            pl.BlockSpec((R, 1), lambda b: (0, 0)),
            pl.BlockSpec((None, F, N), lambda b: (b, 0, 0)),
        ],
        out_specs=[
            pl.BlockSpec((None, R, N), lambda b: (b, 0, 0)),
            pl.BlockSpec((None, 1, N), lambda b: (b, 0, 0)),
        ],
        out_shape=[
            jax.ShapeDtypeStruct((B, R, N), jnp.float32),
            jax.ShapeDtypeStruct((B, 1, N), jnp.int32),
        ],
    )(w2d, b2d, x)
    return val, ids.reshape(B, N)


_MIN32 = -2147483648


def _sort_body(val_hbm, idx_hbm, idxf_hbm,
               valfX, keyAX, valAX, keyBX, valBX, histX, vfX,
               valfY, keyAY, valAY, keyBY, valBY, histY, vfY, bsem):
    c = lax.axis_index("c")
    s = lax.axis_index("s")
    w = s * 2 + c
    lane = lax.iota(jnp.int32, 16)
    lane128 = lane * 128

    def row_loop(ri, _):
        rowX = w * 8 + ri * 2
        rowY = rowX + 1
        pltpu.sync_copy(val_hbm.at[rowX], valfX)
        pltpu.sync_copy(val_hbm.at[rowY], valfY)

        def key_loop(i, _):
            for valf, keyA in ((valfX, keyAX), (valfY, keyAY)):
                x = valf[pl.ds(i * 16, 16)]
                sbits = plsc.bitcast(x, jnp.int32)
                m = lax.shift_right_arithmetic(sbits, 31)
                k = (sbits ^ (m | jnp.int32(_MIN32))) ^ jnp.int32(-1)
                keyA[pl.ds(i * 16, 16)] = k
            return 0

        lax.fori_loop(0, 128, key_loop, 0)

        for p in range(6):
            if p % 2 == 0:
                bufs = ((keyAX, valAX, keyBX, valBX, histX),
                        (keyAY, valAY, keyBY, valBY, histY))
            else:
                bufs = ((keyBX, valBX, keyAX, valAX, histX),
                        (keyBY, valBY, keyAY, valAY, histY))
            sh = 6 * p

            def z_loop(t, _):
                histX[pl.ds(t * 16, 16)] = jnp.zeros((16,), jnp.int32)
                histY[pl.ds(t * 16, 16)] = jnp.zeros((16,), jnp.int32)
                return 0

            lax.fori_loop(0, 64, z_loop, 0)

            def h_loop(i, _, bufs=bufs, sh=sh):
                addr = lane128 + i
                for kSrc, _v, _k, _vd, hist in bufs:
                    kv = plsc.load_gather(kSrc, [addr])
                    d = lax.shift_right_logical(kv, sh) & 63
                    a2 = d * 16 + lane
                    cnt = plsc.load_gather(hist, [a2])
                    plsc.store_scatter(hist, [a2], cnt + 1)
                return 0

            lax.fori_loop(0, 128, h_loop, 0)

            def s_loop(t, runs):
                runX, runY = runs
                hvX = histX[pl.ds(t * 16, 16)]
                hvY = histY[pl.ds(t * 16, 16)]
                incX = plsc.cumsum(hvX)
                incY = plsc.cumsum(hvY)
                histX[pl.ds(t * 16, 16)] = incX - hvX + runX
                histY[pl.ds(t * 16, 16)] = incY - hvY + runY
                return (runX + jnp.sum(hvX), runY + jnp.sum(hvY))

            lax.fori_loop(0, 64, s_loop, (jnp.int32(0), jnp.int32(0)))

            def c_loop(i, _, bufs=bufs, sh=sh, first=(p == 0)):
                addr = lane128 + i
                for kSrc, vSrc, kDst, vDst, hist in bufs:
                    kv = plsc.load_gather(kSrc, [addr])
                    vv = addr if first else plsc.load_gather(vSrc, [addr])
                    d = lax.shift_right_logical(kv, sh) & 63
                    a2 = d * 16 + lane
                    pos = plsc.load_gather(hist, [a2])
                    plsc.store_scatter(hist, [a2], pos + 1)
                    plsc.store_scatter(kDst, [pos], kv)
                    plsc.store_scatter(vDst, [pos], vv)
                return 0

            lax.fori_loop(0, 128, c_loop, 0)

        pltpu.sync_copy(valAX.at[pl.ds(0, PNT)], idx_hbm.at[rowX])
        pltpu.sync_copy(valAY.at[pl.ds(0, PNT)], idx_hbm.at[rowY])

        def cvt_loop(i, _):
            vfX[pl.ds(i * 16, 16)] = valAX[pl.ds(i * 16, 16)].astype(
                jnp.float32)
            vfY[pl.ds(i * 16, 16)] = valAY[pl.ds(i * 16, 16)].astype(
                jnp.float32)
            return 0

        lax.fori_loop(0, PNT // 16, cvt_loop, 0)

        for row, vf in ((rowX, vfX), (rowY, vfY)):
            b = row // R
            r = row % R
            base = b * (NCOPY * R) + r
            for j in range(NCOPY):
                pltpu.async_copy(vf, idxf_hbm.at[base + j * R], bsem)
        for row, vf in ((rowX, vfX), (rowY, vfY)):
            base = (row // R) * (NCOPY * R) + row % R
            for j in range(NCOPY):
                pltpu.make_async_copy(vf, idxf_hbm.at[base], bsem).wait()
        return 0

    lax.fori_loop(0, 4, row_loop, 0)


def _sc_sort(val2d):
    mesh = plsc.VectorSubcoreMesh(core_axis_name="c", subcore_axis_name="s")
    row_scratch = [
        pltpu.VMEM((N,), jnp.float32),
        pltpu.VMEM((N,), jnp.int32),
        pltpu.VMEM((N,), jnp.int32),
        pltpu.VMEM((N,), jnp.int32),
        pltpu.VMEM((N,), jnp.int32),
        pltpu.VMEM((64 * 16,), jnp.int32),
        pltpu.VMEM((PNT,), jnp.float32),
    ]
    return pl.kernel(
        _sort_body,
        out_type=[
            jax.ShapeDtypeStruct((B * R, PNT), jnp.int32),
            jax.ShapeDtypeStruct((B * NCOPY * R, PNT), jnp.float32),
        ],
        mesh=mesh,
        scratch_types=row_scratch + row_scratch + [pltpu.SemaphoreType.DMA],
        compiler_params=pltpu.CompilerParams(needs_layout_passes=False),
    )(val2d)


FB = 4
NBLK = NF_HALF // FB


def _gather_body(x_hbm, idx_hbm, cube_hbm, cab_hbm,
                 idx_v, xr0, xr1, or0, or1, cab_part, cab_all,
                 isem0, isem1, osem0, osem1):
    c = lax.axis_index("c")
    s = lax.axis_index("s")
    w = s * 2 + c
    b = w // 2
    fhalf = w % 2
    row0 = b * F + fhalf * NF_HALF
    xrs = (xr0, xr1)
    ors = (or0, or1)
    isems = (isem0, isem1)
    osems = (osem0, osem1)
    lane = lax.iota(jnp.int32, 16)

    pltpu.sync_copy(idx_hbm.at[b], idx_v)
    blk0 = (row0 // FB)
    pltpu.async_copy(x_hbm.at[blk0], xr0, isem0)

    def blk_loop(i, _):
        for q in range(2):
            bi = i * 2 + q
            blk = blk0 + bi
            xr = xrs[q]
            oblk = ors[q]

            pltpu.make_async_copy(x_hbm.at[blk], xr, isems[q]).wait()
            nbi = jnp.minimum(bi + 1, NBLK - 1)
            pltpu.async_copy(x_hbm.at[blk0 + nbi], xrs[1 - q], isems[1 - q])

            @pl.when(i >= 1)
            def _(oblk=oblk, blk=blk, q=q):
                pltpu.make_async_copy(oblk, cube_hbm.at[blk], osems[q]).wait()

            def r_loop(r, _, xr=xr, oblk=oblk):
                rb = r * PNT
                for c4 in range(8):
                    base = rb + c4 * 64
                    cm0 = jnp.full((16,), -jnp.inf, jnp.float32)
                    cm1 = cm0
                    cm2 = cm0
                    cm3 = cm0
                    for t in range(4):
                        off = base + t * 16
                        iv = idx_v[pl.ds(off, 16)]
                        g0 = plsc.load_gather(xr, [iv])
                        g1 = plsc.load_gather(xr, [iv + N])
                        g2 = plsc.load_gather(xr, [iv + 2 * N])
                        g3 = plsc.load_gather(xr, [iv + 3 * N])
                        oblk[pl.ds(off, 16)] = g0
                        oblk[pl.ds(R * PNT + off, 16)] = g1
                        oblk[pl.ds(2 * R * PNT + off, 16)] = g2
                        oblk[pl.ds(3 * R * PNT + off, 16)] = g3
                        cm0 = jnp.maximum(cm0, g0)
                        cm1 = jnp.maximum(cm1, g1)
                        cm2 = jnp.maximum(cm2, g2)
                        cm3 = jnp.maximum(cm3, g3)
                    pos = r * 128 + c4 * 16
                    cab_part[pl.ds(pos, 16)] = cm0
                    cab_part[pl.ds(2048 + pos, 16)] = cm1
                    cab_part[pl.ds(4096 + pos, 16)] = cm2
                    cab_part[pl.ds(6144 + pos, 16)] = cm3
                return 0

            lax.fori_loop(0, R, r_loop, 0)

            def tr_loop(j, _, bi=bi):
                for fb in range(FB):
                    acc = jnp.full((16,), -jnp.inf, jnp.float32)
                    col = lane * 16 + j * 256 + fb * 2048
                    for l in range(16):
                        v = plsc.load_gather(cab_part, [col + l])
                        acc = jnp.maximum(acc, v)
                    cab_all[bi * FB + fb, pl.ds(j * 16, 16)] = acc
                return 0

            lax.fori_loop(0, 8, tr_loop, 0)

            pltpu.async_copy(oblk, cube_hbm.at[blk], osems[q])
        return 0

    lax.fori_loop(0, NBLK // 2, blk_loop, 0)

    pltpu.make_async_copy(x_hbm.at[blk0 + NBLK - 1], xr0, isem0).wait()
    for q in range(2):
        pltpu.make_async_copy(ors[q], cube_hbm.at[blk0 + NBLK - 2 + q],
                              osems[q]).wait()
    pltpu.sync_copy(cab_all, cab_hbm.at[pl.ds(row0, NF_HALF)])


def _sc_gather(x2d, idxflat):
    mesh = plsc.VectorSubcoreMesh(core_axis_name="c", subcore_axis_name="s")
    return pl.kernel(
        _gather_body,
        out_type=[
            jax.ShapeDtypeStruct((B * F // FB, FB * R * PNT), jnp.float32),
            jax.ShapeDtypeStruct((B * F, R * 8), jnp.float32),
        ],
        mesh=mesh,
        scratch_types=[
            pltpu.VMEM((R * PNT,), jnp.int32),
            pltpu.VMEM((FB * N,), jnp.float32),
            pltpu.VMEM((FB * N,), jnp.float32),
            pltpu.VMEM((FB * R * PNT,), jnp.float32),
            pltpu.VMEM((FB * R * PNT,), jnp.float32),
            pltpu.VMEM((FB * R * 8 * 16,), jnp.float32),
            pltpu.VMEM((NF_HALF, R * 8), jnp.float32),
            pltpu.SemaphoreType.DMA,
            pltpu.SemaphoreType.DMA,
            pltpu.SemaphoreType.DMA,
            pltpu.SemaphoreType.DMA,
        ],
        compiler_params=pltpu.CompilerParams(needs_layout_passes=False),
    )(x2d, idxflat)


def kernel(x, w_sorter, b_sorter, w1, b1, w2, b2, w3, b3, w5, b5):
    val_activa, id_activa = _sorter(x, w_sorter[:, :, 0],
                                    b_sorter.reshape(R, 1))

    idx2d, idxf = _sc_sort(val_activa.reshape(B * R, N))
    idx = idx2d.reshape(B, R, PNT)

    cube, cab = _sc_gather(x.reshape(B * F // FB, FB * N),
                           idx2d.reshape(B, R * PNT))
    sp_cube = cube.reshape(B, F, R, PNT)
    cabins = cab.reshape(B, F, R, 8)
    sp_idx = idxf.reshape(B, NCOPY, R, PNT)
    return (sp_cube, sp_idx, cabins, id_activa)


# speedup vs baseline: 1.0237x; 1.0237x over previous
import functools

import jax
import jax.numpy as jnp
from jax import lax
from jax.experimental import pallas as pl
from jax.experimental.pallas import tpu as pltpu
from jax.experimental.pallas import tpu_sc as plsc

B, F, N = 16, 256, 2048
R, PNT = 16, 512
NF_HALF = F // 2
NCOPY = R + 3


def _sorter_body(w_ref, b_ref, x_ref, val_ref, id_ref):
    w = w_ref[...]
    xb = x_ref[...]
    val = jnp.dot(w, xb, preferred_element_type=jnp.float32)
    val = val + b_ref[...]
    val_ref[...] = val
    mx = jnp.max(val, axis=0, keepdims=True)
    iota = lax.broadcasted_iota(jnp.int32, (R, N), 0)
    ids = jnp.min(jnp.where(val == mx, iota, jnp.int32(2**30)), axis=0,
                  keepdims=True)
    id_ref[...] = ids


def _sorter(x, w2d, b2d):
    val, ids = pl.pallas_call(
        _sorter_body,
        grid=(B,),
        in_specs=[
            pl.BlockSpec((R, F), lambda b: (0, 0)),
            pl.BlockSpec((R, 1), lambda b: (0, 0)),
            pl.BlockSpec((None, F, N), lambda b: (b, 0, 0)),
        ],
        out_specs=[
            pl.BlockSpec((None, R, N), lambda b: (b, 0, 0)),
            pl.BlockSpec((None, 1, N), lambda b: (b, 0, 0)),
        ],
        out_shape=[
            jax.ShapeDtypeStruct((B, R, N), jnp.float32),
            jax.ShapeDtypeStruct((B, 1, N), jnp.int32),
        ],
    )(w2d, b2d, x)
    return val, ids.reshape(B, N)


_MIN32 = -2147483648


def _sort_body(val_hbm, idx_hbm, idxf_hbm, valf, keyA, valA, keyB, valB,
               hist, vf, bsem):
    c = lax.axis_index("c")
    s = lax.axis_index("s")
    w = s * 2 + c
    lane = lax.iota(jnp.int32, 16)

    def row_loop(ri, _):
        row = w * 8 + ri
        pltpu.sync_copy(val_hbm.at[row], valf)

        def key_loop(i, _):
            x = valf[pl.ds(i * 16, 16)]
            sbits = plsc.bitcast(x, jnp.int32)
            m = lax.shift_right_arithmetic(sbits, 31)
            k = (sbits ^ (m | jnp.int32(_MIN32))) ^ jnp.int32(-1)
            keyA[pl.ds(i * 16, 16)] = k
            return 0

        lax.fori_loop(0, 128, key_loop, 0)

        lane128 = lane * 128

        for p in range(6):
            kSrc, vSrc, kDst, vDst = (
                (keyA, valA, keyB, valB) if p % 2 == 0 else
                (keyB, valB, keyA, valA))
            sh = 6 * p

            def z_loop(t, _):
                hist[pl.ds(t * 16, 16)] = jnp.zeros((16,), jnp.int32)
                return 0

            lax.fori_loop(0, 64, z_loop, 0)

            def h_loop(i, _, kSrc=kSrc, sh=sh):
                addr = lane128 + i
                kv = plsc.load_gather(kSrc, [addr])
                d = lax.shift_right_logical(kv, sh) & 63
                a2 = d * 16 + lane
                cnt = plsc.load_gather(hist, [a2])
                plsc.store_scatter(hist, [a2], cnt + 1)
                return 0

            lax.fori_loop(0, 128, h_loop, 0)

            def s_loop(t, run):
                hv = hist[pl.ds(t * 16, 16)]
                inc = plsc.cumsum(hv)
                hist[pl.ds(t * 16, 16)] = inc - hv + run
                return run + jnp.sum(hv)

            lax.fori_loop(0, 64, s_loop, jnp.int32(0))

            def c_loop(i, _, kSrc=kSrc, vSrc=vSrc, kDst=kDst, vDst=vDst,
                       sh=sh, first=(p == 0)):
                addr = lane128 + i
                kv = plsc.load_gather(kSrc, [addr])
                vv = addr if first else plsc.load_gather(vSrc, [addr])
                d = lax.shift_right_logical(kv, sh) & 63
                a2 = d * 16 + lane
                pos = plsc.load_gather(hist, [a2])
                plsc.store_scatter(hist, [a2], pos + 1)
                plsc.store_scatter(kDst, [pos], kv)
                plsc.store_scatter(vDst, [pos], vv)
                return 0

            lax.fori_loop(0, 128, c_loop, 0)

        pltpu.sync_copy(valA.at[pl.ds(0, PNT)], idx_hbm.at[row])

        def cvt_loop(i, _):
            vf[pl.ds(i * 16, 16)] = valA[pl.ds(i * 16, 16)].astype(jnp.float32)
            return 0

        lax.fori_loop(0, PNT // 16, cvt_loop, 0)

        b = row // R
        r = row % R
        base = b * (NCOPY * R) + r
        for j in range(NCOPY):
            pltpu.async_copy(vf, idxf_hbm.at[base + j * R], bsem)
        for j in range(NCOPY):
            pltpu.make_async_copy(vf, idxf_hbm.at[base], bsem).wait()
        return 0

    lax.fori_loop(0, 8, row_loop, 0)


def _sc_sort(val2d):
    mesh = plsc.VectorSubcoreMesh(core_axis_name="c", subcore_axis_name="s")
    return pl.kernel(
        _sort_body,
        out_type=[
            jax.ShapeDtypeStruct((B * R, PNT), jnp.int32),
            jax.ShapeDtypeStruct((B * NCOPY * R, PNT), jnp.float32),
        ],
        mesh=mesh,
        scratch_types=[
            pltpu.VMEM((N,), jnp.float32),
            pltpu.VMEM((N,), jnp.int32),
            pltpu.VMEM((N,), jnp.int32),
            pltpu.VMEM((N,), jnp.int32),
            pltpu.VMEM((N,), jnp.int32),
            pltpu.VMEM((64 * 16,), jnp.int32),
            pltpu.VMEM((PNT,), jnp.float32),
            pltpu.SemaphoreType.DMA,
        ],
        compiler_params=pltpu.CompilerParams(needs_layout_passes=False),
    )(val2d)


FB = 4
NBLK = NF_HALF // FB


def _gather_body(x_hbm, idx_hbm, cube_hbm, cab_hbm,
                 idx_v, xr0, xr1, or0, or1, cab_part, cab_all,
                 isem0, isem1, osem0, osem1):
    c = lax.axis_index("c")
    s = lax.axis_index("s")
    w = s * 2 + c
    b = w // 2
    fhalf = w % 2
    row0 = b * F + fhalf * NF_HALF
    xrs = (xr0, xr1)
    ors = (or0, or1)
    isems = (isem0, isem1)
    osems = (osem0, osem1)
    lane = lax.iota(jnp.int32, 16)

    pltpu.sync_copy(idx_hbm.at[b], idx_v)
    blk0 = (row0 // FB)
    pltpu.async_copy(x_hbm.at[blk0], xr0, isem0)

    def blk_loop(i, _):
        for q in range(2):
            bi = i * 2 + q
            blk = blk0 + bi
            xr = xrs[q]
            oblk = ors[q]

            pltpu.make_async_copy(x_hbm.at[blk], xr, isems[q]).wait()
            nbi = jnp.minimum(bi + 1, NBLK - 1)
            pltpu.async_copy(x_hbm.at[blk0 + nbi], xrs[1 - q], isems[1 - q])

            @pl.when(i >= 1)
            def _(oblk=oblk, blk=blk, q=q):
                pltpu.make_async_copy(oblk, cube_hbm.at[blk], osems[q]).wait()

            def r_loop(r, _, xr=xr, oblk=oblk):
                rb = r * PNT
                for c4 in range(8):
                    base = rb + c4 * 64
                    cm0 = jnp.full((16,), -jnp.inf, jnp.float32)
                    cm1 = cm0
                    cm2 = cm0
                    cm3 = cm0
                    for t in range(4):
                        off = base + t * 16
                        iv = idx_v[pl.ds(off, 16)]
                        g0 = plsc.load_gather(xr, [iv])
                        g1 = plsc.load_gather(xr, [iv + N])
                        g2 = plsc.load_gather(xr, [iv + 2 * N])
                        g3 = plsc.load_gather(xr, [iv + 3 * N])
                        oblk[pl.ds(off, 16)] = g0
                        oblk[pl.ds(R * PNT + off, 16)] = g1
                        oblk[pl.ds(2 * R * PNT + off, 16)] = g2
                        oblk[pl.ds(3 * R * PNT + off, 16)] = g3
                        cm0 = jnp.maximum(cm0, g0)
                        cm1 = jnp.maximum(cm1, g1)
                        cm2 = jnp.maximum(cm2, g2)
                        cm3 = jnp.maximum(cm3, g3)
                    pos = r * 128 + c4 * 16
                    cab_part[pl.ds(pos, 16)] = cm0
                    cab_part[pl.ds(2048 + pos, 16)] = cm1
                    cab_part[pl.ds(4096 + pos, 16)] = cm2
                    cab_part[pl.ds(6144 + pos, 16)] = cm3
                return 0

            lax.fori_loop(0, R, r_loop, 0)

            def tr_loop(j, _, bi=bi):
                for fb in range(FB):
                    acc = jnp.full((16,), -jnp.inf, jnp.float32)
                    col = lane * 16 + j * 256 + fb * 2048
                    for l in range(16):
                        v = plsc.load_gather(cab_part, [col + l])
                        acc = jnp.maximum(acc, v)
                    cab_all[bi * FB + fb, pl.ds(j * 16, 16)] = acc
                return 0

            lax.fori_loop(0, 8, tr_loop, 0)

            pltpu.async_copy(oblk, cube_hbm.at[blk], osems[q])
        return 0

    lax.fori_loop(0, NBLK // 2, blk_loop, 0)

    pltpu.make_async_copy(x_hbm.at[blk0 + NBLK - 1], xr0, isem0).wait()
    for q in range(2):
        pltpu.make_async_copy(ors[q], cube_hbm.at[blk0 + NBLK - 2 + q],
                              osems[q]).wait()
    pltpu.sync_copy(cab_all, cab_hbm.at[pl.ds(row0, NF_HALF)])


def _sc_gather(x2d, idxflat):
    mesh = plsc.VectorSubcoreMesh(core_axis_name="c", subcore_axis_name="s")
    return pl.kernel(
        _gather_body,
        out_type=[
            jax.ShapeDtypeStruct((B * F // FB, FB * R * PNT), jnp.float32),
            jax.ShapeDtypeStruct((B * F, R * 8), jnp.float32),
        ],
        mesh=mesh,
        scratch_types=[
            pltpu.VMEM((R * PNT,), jnp.int32),
            pltpu.VMEM((FB * N,), jnp.float32),
            pltpu.VMEM((FB * N,), jnp.float32),
            pltpu.VMEM((FB * R * PNT,), jnp.float32),
            pltpu.VMEM((FB * R * PNT,), jnp.float32),
            pltpu.VMEM((FB * R * 8 * 16,), jnp.float32),
            pltpu.VMEM((NF_HALF, R * 8), jnp.float32),
            pltpu.SemaphoreType.DMA,
            pltpu.SemaphoreType.DMA,
            pltpu.SemaphoreType.DMA,
            pltpu.SemaphoreType.DMA,
        ],
        compiler_params=pltpu.CompilerParams(needs_layout_passes=False),
    )(x2d, idxflat)


def kernel(x, w_sorter, b_sorter, w1, b1, w2, b2, w3, b3, w5, b5):
    val_activa, id_activa = _sorter(x, w_sorter[:, :, 0],
                                    b_sorter.reshape(R, 1))

    idx2d, idxf = _sc_sort(val_activa.reshape(B * R, N))
    idx = idx2d.reshape(B, R, PNT)

    cube, cab = _sc_gather(x.reshape(B * F // FB, FB * N),
                           idx2d.reshape(B, R * PNT))
    sp_cube = cube.reshape(B, F, R, PNT)
    cabins = cab.reshape(B, F, R, 8)
    sp_idx = idxf.reshape(B, NCOPY, R, PNT)
    return (sp_cube, sp_idx, cabins, id_activa)
